# baseline (speedup 1.0000x reference)
import functools

import jax
import jax.numpy as jnp
from jax import lax
from jax.experimental import pallas as pl
from jax.experimental.pallas import tpu as pltpu
from jax.experimental.pallas import tpu_sc as plsc

N = 50000
E = 800000
F_IN = 8
T_IN = 12
T_OUT = 12
F96 = F_IN * T_OUT
FP = 32
NPASS = F96 // FP

LANES = 16
NC = 2
NS = 16
NW = NC * NS

ROW = 128
EPAD = 819200
EROWS = EPAD // ROW
RPW = EROWS // NW
RB = 2
NJJ = RPW // RB
RB1 = 4
NJJ1 = RPW // RB1

NPAD = 50048
NSTRIPE = NPAD // NS
NCHUNK = 136
NK = NSTRIPE // NCHUNK

NPAD_DEG = NPAD
DSTRIPE = NPAD_DEG // NS


def _sc_mesh():
    return plsc.VectorSubcoreMesh(core_axis_name="c", subcore_axis_name="s")


def _k1_body(dst_hbm, ew_hbm, out0_hbm, out1_hbm, deg_sp, ibuf, wbuf, zbuf, obuf):
    c = lax.axis_index("c")
    s = lax.axis_index("s")
    wid = s * NC + c

    def _z(i, _):
        zbuf[pl.ds(i * LANES, LANES)] = jnp.zeros((LANES,), jnp.float32)
        return ()
    lax.fori_loop(0, DSTRIPE // LANES, _z, ())
    pltpu.sync_copy(zbuf, deg_sp.at[pl.ds(s * DSTRIPE, DSTRIPE)])
    plsc.subcore_barrier()

    base = wid * RPW

    def _edges(jj, _):
        row0 = base + jj * RB1
        pltpu.sync_copy(dst_hbm.at[pl.ds(row0, RB1)], ibuf)
        pltpu.sync_copy(ew_hbm.at[pl.ds(row0, RB1)], wbuf)

        def _r(r, _):
            pltpu.sync_copy(wbuf.at[r], deg_sp.at[ibuf.at[r]], add=True)
            return ()
        lax.fori_loop(0, RB1, _r, ())
        return ()
    lax.fori_loop(0, NJJ1, _edges, ())
    plsc.subcore_barrier()

    pltpu.sync_copy(deg_sp.at[pl.ds(s * DSTRIPE, DSTRIPE)], obuf)

    @pl.when(c == 0)
    def _():
        pltpu.sync_copy(obuf, out0_hbm.at[pl.ds(s * DSTRIPE, DSTRIPE)])

    @pl.when(c == 1)
    def _():
        pltpu.sync_copy(obuf, out1_hbm.at[pl.ds(s * DSTRIPE, DSTRIPE)])


def _k1(dst2d, ew2d):
    f = functools.partial(
        pl.kernel,
        out_type=[jax.ShapeDtypeStruct((NPAD_DEG,), jnp.float32),
                  jax.ShapeDtypeStruct((NPAD_DEG,), jnp.float32)],
        mesh=_sc_mesh(),
        scratch_types=[
            pltpu.VMEM_SHARED((NPAD_DEG,), jnp.float32),
            pltpu.VMEM((RB1, ROW), jnp.int32),
            pltpu.VMEM((RB1, ROW), jnp.float32),
            pltpu.VMEM((DSTRIPE,), jnp.float32),
            pltpu.VMEM((DSTRIPE,), jnp.float32),
        ],
    )(_k1_body)
    return f(dst2d, ew2d)


_B2 = 400
_G2 = N // _B2


def _k2_body(degT_ref, x_ref, dinv_ref, x0_ref, x1_ref, x2_ref):
    d = degT_ref[:, 0] + degT_ref[:, 1] + 1.0
    dinv = jnp.where(d > 0, lax.rsqrt(d), 0.0)[:, None]
    dinv_ref[...] = dinv
    xs = x_ref[...] * dinv
    x0_ref[...] = xs[:, 0 * FP:1 * FP]
    x1_ref[...] = xs[:, 1 * FP:2 * FP]
    x2_ref[...] = xs[:, 2 * FP:3 * FP]


def _k2(degT, x_pm):
    return pl.pallas_call(
        _k2_body,
        grid=(_G2,),
        in_specs=[
            pl.BlockSpec((_B2, 2), lambda i: (i, 0)),
            pl.BlockSpec((_B2, F96), lambda i: (i, 0)),
        ],
        out_specs=[
            pl.BlockSpec((_B2, 1), lambda i: (i, 0)),
            pl.BlockSpec((_B2, FP), lambda i: (i, 0)),
            pl.BlockSpec((_B2, FP), lambda i: (i, 0)),
            pl.BlockSpec((_B2, FP), lambda i: (i, 0)),
        ],
        out_shape=[
            jax.ShapeDtypeStruct((N, 1), jnp.float32),
            jax.ShapeDtypeStruct((N, FP), jnp.float32),
            jax.ShapeDtypeStruct((N, FP), jnp.float32),
            jax.ShapeDtypeStruct((N, FP), jnp.float32),
        ],
    )(degT, x_pm)


def _k3_body(src_hbm, dst_hbm, ew_hbm, x0_hbm, x1_hbm, x2_hbm, out_hbm,
             acc_sp, sbuf, dbuf, wbuf, rows, zb, ob, gsem):
    c = lax.axis_index("c")
    s = lax.axis_index("s")
    wid = s * NC + c
    base = wid * RPW

    def _z(i, _):
        zb[i, pl.ds(0, LANES)] = jnp.zeros((LANES,), jnp.float32)
        zb[i, pl.ds(LANES, LANES)] = jnp.zeros((LANES,), jnp.float32)
        return ()
    lax.fori_loop(0, NCHUNK, _z, ())

    for p, x_hbm in enumerate((x0_hbm, x1_hbm, x2_hbm)):
        def _zero(k, _):
            pltpu.sync_copy(zb, acc_sp.at[pl.ds(s * NSTRIPE + k * NCHUNK, NCHUNK), :])
            return ()
        lax.fori_loop(0, NK, _zero, ())
        plsc.subcore_barrier()

        def _edges(jj, _):
            row0 = base + jj * RB1
            pltpu.sync_copy(src_hbm.at[pl.ds(row0, RB1)], sbuf)
            pltpu.sync_copy(dst_hbm.at[pl.ds(row0, RB1)], dbuf)
            pltpu.sync_copy(ew_hbm.at[pl.ds(row0, RB1)], wbuf)

            def _r(r, _):
                pltpu.async_copy(x_hbm.at[sbuf.at[r]], rows, gsem).wait()

                def _g(g, _):
                    ew16 = wbuf[r, pl.ds(g * LANES, LANES)]
                    for k in range(LANES):
                        bval = ew16[jnp.full((LANES,), k, jnp.int32)]
                        e = g * LANES + k
                        rows[e, pl.ds(0, LANES)] = rows[e, pl.ds(0, LANES)] * bval
                        rows[e, pl.ds(LANES, LANES)] = (
                            rows[e, pl.ds(LANES, LANES)] * bval)
                    return ()
                lax.fori_loop(0, ROW // LANES, _g, ())

                pltpu.sync_copy(rows, acc_sp.at[dbuf.at[r]], add=True)
                return ()
            lax.fori_loop(0, RB1, _r, ())
            return ()
        lax.fori_loop(0, NJJ1, _edges, ())
        plsc.subcore_barrier()

        def _rd(k, _):
            row0 = s * NSTRIPE + k * NCHUNK
            pltpu.sync_copy(acc_sp.at[pl.ds(row0, NCHUNK), :], ob)

            @pl.when(c == 0)
            def _():
                pltpu.sync_copy(ob, out_hbm.at[0, p, pl.ds(row0, NCHUNK), :])

            @pl.when(c == 1)
            def _():
                pltpu.sync_copy(ob, out_hbm.at[1, p, pl.ds(row0, NCHUNK), :])
            return ()
        lax.fori_loop(0, NK, _rd, ())
        if p != NPASS - 1:
            plsc.subcore_barrier()


def _k3(src2d, dst2d, ew2d, x0, x1, x2):
    f = functools.partial(
        pl.kernel,
        out_type=jax.ShapeDtypeStruct((NC, NPASS, NPAD, FP), jnp.float32),
        mesh=_sc_mesh(),
        scratch_types=[
            pltpu.VMEM_SHARED((NPAD, FP), jnp.float32),
            pltpu.VMEM((RB1, ROW), jnp.int32),
            pltpu.VMEM((RB1, ROW), jnp.int32),
            pltpu.VMEM((RB1, ROW), jnp.float32),
            pltpu.VMEM((ROW, FP), jnp.float32),
            pltpu.VMEM((NCHUNK, FP), jnp.float32),
            pltpu.VMEM((NCHUNK, FP), jnp.float32),
            pltpu.SemaphoreType.DMA,
        ],
        compiler_params=pltpu.CompilerParams(use_tc_tiling_on_sc=False),
    )(_k3_body)
    return f(src2d, dst2d, ew2d, x0, x1, x2)


_B4 = 400
_G4 = N // _B4


def _split(v):
    hi = v.astype(jnp.bfloat16)
    lo = (v - hi.astype(jnp.float32)).astype(jnp.bfloat16)
    return hi, lo


def _dot3(x, whi_lo):
    whi, wlo = whi_lo
    xhi, xlo = _split(x)
    f32 = jnp.float32
    return (jnp.dot(xhi, whi, preferred_element_type=f32)
            + jnp.dot(xhi, wlo, preferred_element_type=f32)
            + jnp.dot(xlo, whi, preferred_element_type=f32))


def _k4_body(a_ref, x0_ref, x1_ref, x2_ref, dinv_ref, wz_hi, wz_lo,
             wh_hi, wh_lo, cbz_ref, cbh_ref, pw_hi, pw_lo, lw_hi, lw_lo,
             lb_ref, out_ref):
    dinv = dinv_ref[...]
    parts = [dinv * (a_ref[0, p] + a_ref[1, p] + xr[...])
             for p, xr in enumerate((x0_ref, x1_ref, x2_ref))]
    y = jnp.concatenate(parts, axis=1)
    yhi, ylo = _split(y)
    f32 = jnp.float32

    def dot3(whi_ref, wlo_ref):
        return (jnp.dot(yhi, whi_ref[...], preferred_element_type=f32)
                + jnp.dot(yhi, wlo_ref[...], preferred_element_type=f32)
                + jnp.dot(ylo, whi_ref[...], preferred_element_type=f32))

    s = dot3(wz_hi, wz_lo) + cbz_ref[...]
    hh = dot3(wh_hi, wh_lo) + cbh_ref[...]
    hp = 0.5 * (1.0 - jnp.tanh(0.5 * s)) * jnp.tanh(hh)
    hacc = _dot3(hp, (pw_hi[...], pw_lo[...]))
    h = jax.nn.relu(hacc)
    out_ref[...] = _dot3(h, (lw_hi[...], lw_lo[...])) + lb_ref[...]


def _k4(a, x0, x1, x2, dinv, wz, wh, cbz_t, cbh_t, pw, lw, lb):
    zero = lambda i: (0, 0)
    return pl.pallas_call(
        _k4_body,
        grid=(_G4,),
        in_specs=[
            pl.BlockSpec((NC, NPASS, _B4, FP), lambda i: (0, 0, i, 0)),
            pl.BlockSpec((_B4, FP), lambda i: (i, 0)),
            pl.BlockSpec((_B4, FP), lambda i: (i, 0)),
            pl.BlockSpec((_B4, FP), lambda i: (i, 0)),
            pl.BlockSpec((_B4, 1), lambda i: (i, 0)),
            pl.BlockSpec((F96, 144), zero),
            pl.BlockSpec((F96, 144), zero),
            pl.BlockSpec((F96, 144), zero),
            pl.BlockSpec((F96, 144), zero),
            pl.BlockSpec((1, 144), zero),
            pl.BlockSpec((1, 144), zero),
            pl.BlockSpec((144, T_IN), zero),
            pl.BlockSpec((144, T_IN), zero),
            pl.BlockSpec((T_IN, T_OUT), zero),
            pl.BlockSpec((T_IN, T_OUT), zero),
            pl.BlockSpec((1, T_OUT), zero),
        ],
        out_specs=pl.BlockSpec((_B4, T_OUT), lambda i: (i, 0)),
        out_shape=jax.ShapeDtypeStruct((N, T_OUT), jnp.float32),
    )(a, x0, x1, x2, dinv, *wz, *wh, cbz_t, cbh_t, *pw, *lw, lb)


def kernel(x, edge_weight, attention, Wz, bz, Wr, br, Wh, bh, lzW, lzb,
           lrW, lrb, lhW, lhb, linW, linb, edge_index):
    src = edge_index[0]
    dst = edge_index[1]

    pad = EPAD - E
    src2d = jnp.pad(src, (0, pad)).reshape(EROWS, ROW)
    dst2d = jnp.pad(dst, (0, pad)).reshape(EROWS, ROW)
    ew2d = jnp.pad(edge_weight, (0, pad)).reshape(EROWS, ROW)

    x_pm = x.transpose(0, 2, 1).reshape(N, F96)

    deg0, deg1 = _k1(dst2d, ew2d)
    degT = jnp.stack([deg0[:N], deg1[:N]], axis=1)

    dinv, x0, x1, x2 = _k2(degT, x_pm)

    a = _k3(src2d, dst2d, ew2d, x0, x1, x2)

    Az = lzW[:, :T_IN].T
    Ah = lhW[:, :T_IN].T
    Mz = Wz @ Az
    Mh = Wh @ Ah
    cbz = bz @ Az + lzb
    cbh = bh @ Ah + lhb
    probs = jax.nn.softmax(attention)
    BMz = jnp.einsum('pq,fk->pfqk', jnp.eye(T_OUT, dtype=jnp.float32), Mz
                     ).reshape(F96, T_OUT * T_IN)
    BMh = jnp.einsum('pq,fk->pfqk', jnp.eye(T_OUT, dtype=jnp.float32), Mh
                     ).reshape(F96, T_OUT * T_IN)
    cbz_t = jnp.tile(cbz, T_OUT)[None, :]
    cbh_t = jnp.tile(cbh, T_OUT)[None, :]
    eyeT = jnp.eye(T_IN, dtype=jnp.float32)
    pw = (probs[:, None, None] * eyeT[None]).reshape(T_OUT * T_IN, T_IN)
    lwt = linW.T
    lb = linb[None, :]

    return _k4(a, x0, x1, x2, dinv, _split(BMz), _split(BMh), cbz_t, cbh_t,
               _split(pw), _split(lwt), lb)


# speedup vs baseline: 1.6417x; 1.6417x over previous
import functools

import jax
import jax.numpy as jnp
from jax import lax
from jax.experimental import pallas as pl
from jax.experimental.pallas import tpu as pltpu
from jax.experimental.pallas import tpu_sc as plsc

N = 50000
E = 800000
F_IN = 8
T_IN = 12
T_OUT = 12
F96 = F_IN * T_OUT
FP = 32
NPASS = F96 // FP

LANES = 16
NC = 2
NS = 16
NW = NC * NS

ROW = 128
EPAD = 819200
EROWS = EPAD // ROW
RPW = EROWS // NW
RB = 4
NJJ = RPW // RB

NPAD = 50048
NSTRIPE = NPAD // NS
NCHUNK = 136
NK = NSTRIPE // NCHUNK

NPAD_DEG = NPAD
DSTRIPE = NPAD_DEG // NS


def _sc_mesh():
    return plsc.VectorSubcoreMesh(core_axis_name="c", subcore_axis_name="s")


def _k1_body(dst_hbm, ew_hbm, out0_hbm, out1_hbm, deg_sp, ibuf, wbuf, zbuf, obuf):
    c = lax.axis_index("c")
    s = lax.axis_index("s")
    wid = s * NC + c

    def _z(i, _):
        zbuf[pl.ds(i * LANES, LANES)] = jnp.zeros((LANES,), jnp.float32)
        return ()
    lax.fori_loop(0, DSTRIPE // LANES, _z, ())
    pltpu.sync_copy(zbuf, deg_sp.at[pl.ds(s * DSTRIPE, DSTRIPE)])
    plsc.subcore_barrier()

    base = wid * RPW

    def _edges(jj, _):
        row0 = base + jj * RB
        pltpu.sync_copy(dst_hbm.at[pl.ds(row0, RB)], ibuf)
        pltpu.sync_copy(ew_hbm.at[pl.ds(row0, RB)], wbuf)

        def _r(r, _):
            pltpu.sync_copy(wbuf.at[r], deg_sp.at[ibuf.at[r]], add=True)
            return ()
        lax.fori_loop(0, RB, _r, ())
        return ()
    lax.fori_loop(0, NJJ, _edges, ())
    plsc.subcore_barrier()

    pltpu.sync_copy(deg_sp.at[pl.ds(s * DSTRIPE, DSTRIPE)], obuf)

    @pl.when(c == 0)
    def _():
        pltpu.sync_copy(obuf, out0_hbm.at[pl.ds(s * DSTRIPE, DSTRIPE)])

    @pl.when(c == 1)
    def _():
        pltpu.sync_copy(obuf, out1_hbm.at[pl.ds(s * DSTRIPE, DSTRIPE)])


def _k1(dst2d, ew2d):
    f = functools.partial(
        pl.kernel,
        out_type=[jax.ShapeDtypeStruct((NPAD_DEG,), jnp.float32),
                  jax.ShapeDtypeStruct((NPAD_DEG,), jnp.float32)],
        mesh=_sc_mesh(),
        scratch_types=[
            pltpu.VMEM_SHARED((NPAD_DEG,), jnp.float32),
            pltpu.VMEM((RB, ROW), jnp.int32),
            pltpu.VMEM((RB, ROW), jnp.float32),
            pltpu.VMEM((DSTRIPE,), jnp.float32),
            pltpu.VMEM((DSTRIPE,), jnp.float32),
        ],
    )(_k1_body)
    return f(dst2d, ew2d)


_B2 = 400
_G2 = N // _B2


def _k2_body(degT_ref, x_ref, dinv_ref, x0_ref, x1_ref, x2_ref):
    d = degT_ref[:, 0] + degT_ref[:, 1] + 1.0
    dinv = jnp.where(d > 0, lax.rsqrt(d), 0.0)[:, None]
    dinv_ref[...] = dinv
    xs = x_ref[...] * dinv
    x0_ref[...] = xs[:, 0 * FP:1 * FP]
    x1_ref[...] = xs[:, 1 * FP:2 * FP]
    x2_ref[...] = xs[:, 2 * FP:3 * FP]


def _k2(degT, x_pm):
    return pl.pallas_call(
        _k2_body,
        grid=(_G2,),
        in_specs=[
            pl.BlockSpec((_B2, 2), lambda i: (i, 0)),
            pl.BlockSpec((_B2, F96), lambda i: (i, 0)),
        ],
        out_specs=[
            pl.BlockSpec((_B2, 1), lambda i: (i, 0)),
            pl.BlockSpec((_B2, FP), lambda i: (i, 0)),
            pl.BlockSpec((_B2, FP), lambda i: (i, 0)),
            pl.BlockSpec((_B2, FP), lambda i: (i, 0)),
        ],
        out_shape=[
            jax.ShapeDtypeStruct((N, 1), jnp.float32),
            jax.ShapeDtypeStruct((N, FP), jnp.float32),
            jax.ShapeDtypeStruct((N, FP), jnp.float32),
            jax.ShapeDtypeStruct((N, FP), jnp.float32),
        ],
    )(degT, x_pm)


def _k3_body(src_hbm, dst_hbm, ew_hbm, x0_hbm, x1_hbm, x2_hbm, out_hbm,
             acc_sp, sbuf, dbuf, wbuf, rows0, rows1, rows2, rows3,
             zb, ob, esem, gsem, ssem):
    rows_b = (rows0, rows1, rows2, rows3)
    c = lax.axis_index("c")
    s = lax.axis_index("s")
    wid = s * NC + c
    base = wid * RPW

    def _z(i, _):
        zb[i, pl.ds(0, LANES)] = jnp.zeros((LANES,), jnp.float32)
        zb[i, pl.ds(LANES, LANES)] = jnp.zeros((LANES,), jnp.float32)
        return ()
    lax.fori_loop(0, NCHUNK, _z, ())

    for p, x_hbm in enumerate((x0_hbm, x1_hbm, x2_hbm)):
        def _zero(k, _):
            pltpu.sync_copy(zb, acc_sp.at[pl.ds(s * NSTRIPE + k * NCHUNK, NCHUNK), :])
            return ()
        lax.fori_loop(0, NK, _zero, ())
        plsc.subcore_barrier()

        def _edges(jj, _):
            row0 = base + jj * RB
            pltpu.sync_copy(src_hbm.at[pl.ds(row0, RB)], sbuf)
            pltpu.sync_copy(dst_hbm.at[pl.ds(row0, RB)], dbuf)
            pltpu.sync_copy(ew_hbm.at[pl.ds(row0, RB)], wbuf)

            gd = [pltpu.async_copy(x_hbm.at[sbuf.at[r]], rows_b[r], gsem)
                  for r in range(RB)]
            for d in gd:
                d.wait()
            for r in range(RB):
                rows = rows_b[r]

                def _g(g, _, rows=rows, r=r):
                    ew16 = wbuf[r, pl.ds(g * LANES, LANES)]
                    for k in range(LANES):
                        bval = ew16[jnp.full((LANES,), k, jnp.int32)]
                        e = g * LANES + k
                        rows[e, pl.ds(0, LANES)] = rows[e, pl.ds(0, LANES)] * bval
                        rows[e, pl.ds(LANES, LANES)] = (
                            rows[e, pl.ds(LANES, LANES)] * bval)
                    return ()
                lax.fori_loop(0, ROW // LANES, _g, ())

                pltpu.sync_copy(rows, acc_sp.at[dbuf.at[r]], add=True)
            return ()
        lax.fori_loop(0, NJJ, _edges, ())
        plsc.subcore_barrier()

        def _rd(k, _):
            row0 = s * NSTRIPE + k * NCHUNK
            pltpu.sync_copy(acc_sp.at[pl.ds(row0, NCHUNK), :], ob)

            @pl.when(c == 0)
            def _():
                pltpu.sync_copy(ob, out_hbm.at[0, p, pl.ds(row0, NCHUNK), :])

            @pl.when(c == 1)
            def _():
                pltpu.sync_copy(ob, out_hbm.at[1, p, pl.ds(row0, NCHUNK), :])
            return ()
        lax.fori_loop(0, NK, _rd, ())
        if p != NPASS - 1:
            plsc.subcore_barrier()


def _k3(src2d, dst2d, ew2d, x0, x1, x2):
    f = functools.partial(
        pl.kernel,
        out_type=jax.ShapeDtypeStruct((NC, NPASS, NPAD, FP), jnp.float32),
        mesh=_sc_mesh(),
        scratch_types=[
            pltpu.VMEM_SHARED((NPAD, FP), jnp.float32),
            pltpu.VMEM((RB, ROW), jnp.int32),
            pltpu.VMEM((RB, ROW), jnp.int32),
            pltpu.VMEM((RB, ROW), jnp.float32),
            pltpu.VMEM((ROW, FP), jnp.float32),
            pltpu.VMEM((ROW, FP), jnp.float32),
            pltpu.VMEM((ROW, FP), jnp.float32),
            pltpu.VMEM((ROW, FP), jnp.float32),
            pltpu.VMEM((NCHUNK, FP), jnp.float32),
            pltpu.VMEM((NCHUNK, FP), jnp.float32),
            pltpu.SemaphoreType.DMA,
            pltpu.SemaphoreType.DMA,
            pltpu.SemaphoreType.DMA,
        ],
        compiler_params=pltpu.CompilerParams(use_tc_tiling_on_sc=False),
    )(_k3_body)
    return f(src2d, dst2d, ew2d, x0, x1, x2)


_B4 = 400
_G4 = N // _B4


def _split(v):
    hi = v.astype(jnp.bfloat16)
    lo = (v - hi.astype(jnp.float32)).astype(jnp.bfloat16)
    return hi, lo


def _dot3(x, whi_lo):
    whi, wlo = whi_lo
    xhi, xlo = _split(x)
    f32 = jnp.float32
    return (jnp.dot(xhi, whi, preferred_element_type=f32)
            + jnp.dot(xhi, wlo, preferred_element_type=f32)
            + jnp.dot(xlo, whi, preferred_element_type=f32))


def _k4_body(a_ref, x0_ref, x1_ref, x2_ref, dinv_ref, wz_hi, wz_lo,
             wh_hi, wh_lo, cbz_ref, cbh_ref, pw_hi, pw_lo, lw_hi, lw_lo,
             lb_ref, out_ref):
    dinv = dinv_ref[...]
    parts = [dinv * (a_ref[0, p] + a_ref[1, p] + xr[...])
             for p, xr in enumerate((x0_ref, x1_ref, x2_ref))]
    y = jnp.concatenate(parts, axis=1)
    yhi, ylo = _split(y)
    f32 = jnp.float32

    def dot3(whi_ref, wlo_ref):
        return (jnp.dot(yhi, whi_ref[...], preferred_element_type=f32)
                + jnp.dot(yhi, wlo_ref[...], preferred_element_type=f32)
                + jnp.dot(ylo, whi_ref[...], preferred_element_type=f32))

    s = dot3(wz_hi, wz_lo) + cbz_ref[...]
    hh = dot3(wh_hi, wh_lo) + cbh_ref[...]
    hp = 0.5 * (1.0 - jnp.tanh(0.5 * s)) * jnp.tanh(hh)
    hacc = _dot3(hp, (pw_hi[...], pw_lo[...]))
    h = jax.nn.relu(hacc)
    out_ref[...] = _dot3(h, (lw_hi[...], lw_lo[...])) + lb_ref[...]


def _k4(a, x0, x1, x2, dinv, wz, wh, cbz_t, cbh_t, pw, lw, lb):
    zero = lambda i: (0, 0)
    return pl.pallas_call(
        _k4_body,
        grid=(_G4,),
        in_specs=[
            pl.BlockSpec((NC, NPASS, _B4, FP), lambda i: (0, 0, i, 0)),
            pl.BlockSpec((_B4, FP), lambda i: (i, 0)),
            pl.BlockSpec((_B4, FP), lambda i: (i, 0)),
            pl.BlockSpec((_B4, FP), lambda i: (i, 0)),
            pl.BlockSpec((_B4, 1), lambda i: (i, 0)),
            pl.BlockSpec((F96, 144), zero),
            pl.BlockSpec((F96, 144), zero),
            pl.BlockSpec((F96, 144), zero),
            pl.BlockSpec((F96, 144), zero),
            pl.BlockSpec((1, 144), zero),
            pl.BlockSpec((1, 144), zero),
            pl.BlockSpec((144, T_IN), zero),
            pl.BlockSpec((144, T_IN), zero),
            pl.BlockSpec((T_IN, T_OUT), zero),
            pl.BlockSpec((T_IN, T_OUT), zero),
            pl.BlockSpec((1, T_OUT), zero),
        ],
        out_specs=pl.BlockSpec((_B4, T_OUT), lambda i: (i, 0)),
        out_shape=jax.ShapeDtypeStruct((N, T_OUT), jnp.float32),
    )(a, x0, x1, x2, dinv, *wz, *wh, cbz_t, cbh_t, *pw, *lw, lb)


def kernel(x, edge_weight, attention, Wz, bz, Wr, br, Wh, bh, lzW, lzb,
           lrW, lrb, lhW, lhb, linW, linb, edge_index):
    src = edge_index[0]
    dst = edge_index[1]

    pad = EPAD - E
    spread = jnp.arange(pad, dtype=jnp.int32) % N
    src2d = jnp.concatenate([src, spread]).reshape(EROWS, ROW)
    dst2d = jnp.concatenate([dst, spread]).reshape(EROWS, ROW)
    ew2d = jnp.pad(edge_weight, (0, pad)).reshape(EROWS, ROW)

    x_pm = x.transpose(0, 2, 1).reshape(N, F96)

    deg0, deg1 = _k1(dst2d, ew2d)
    degT = jnp.stack([deg0[:N], deg1[:N]], axis=1)

    dinv, x0, x1, x2 = _k2(degT, x_pm)

    a = _k3(src2d, dst2d, ew2d, x0, x1, x2)

    Az = lzW[:, :T_IN].T
    Ah = lhW[:, :T_IN].T
    Mz = Wz @ Az
    Mh = Wh @ Ah
    cbz = bz @ Az + lzb
    cbh = bh @ Ah + lhb
    probs = jax.nn.softmax(attention)
    BMz = jnp.einsum('pq,fk->pfqk', jnp.eye(T_OUT, dtype=jnp.float32), Mz
                     ).reshape(F96, T_OUT * T_IN)
    BMh = jnp.einsum('pq,fk->pfqk', jnp.eye(T_OUT, dtype=jnp.float32), Mh
                     ).reshape(F96, T_OUT * T_IN)
    cbz_t = jnp.tile(cbz, T_OUT)[None, :]
    cbh_t = jnp.tile(cbh, T_OUT)[None, :]
    eyeT = jnp.eye(T_IN, dtype=jnp.float32)
    pw = (probs[:, None, None] * eyeT[None]).reshape(T_OUT * T_IN, T_IN)
    lwt = linW.T
    lb = linb[None, :]

    return _k4(a, x0, x1, x2, dinv, _split(BMz), _split(BMh), cbz_t, cbh_t,
               _split(pw), _split(lwt), lb)
